# baseline (speedup 1.0000x reference)
import functools

import jax
import jax.numpy as jnp
from jax import lax
from jax.experimental import pallas as pl
from jax.experimental.pallas import tpu as pltpu
from jax.experimental.pallas import tpu_sc as plsc

_N_NODES = 100000
_N_EDGES = 6400000

_NC = 2
_NS = 16
_NW = _NC * _NS

_NBINS = 102400
_BIN_PER = _NBINS // _NW
_E_PER = _N_EDGES // _NW
_W = 4000
_NWIN = _E_PER // _W
_LANES = 16

_mesh = plsc.VectorSubcoreMesh(
    core_axis_name="c", subcore_axis_name="s", num_cores=_NC, num_subcores=_NS
)

_params = pltpu.CompilerParams(needs_layout_passes=False)


def _worker_id():
    return lax.axis_index("s") * _NC + lax.axis_index("c")


@functools.partial(
    pl.kernel,
    out_type=jax.ShapeDtypeStruct((_NW * _NBINS,), jnp.int32),
    mesh=_mesh,
    compiler_params=_params,
    scratch_types=[
        pltpu.VMEM((_NBINS,), jnp.int32),
        pltpu.VMEM((_W,), jnp.int32),
        pltpu.VMEM((_W,), jnp.int32),
        pltpu.SemaphoreType.DMA((2,)),
    ],
)
def _hist_kernel(target_hbm, part_hbm, hist_v, idx0_v, idx1_v, sems):
    wid = _worker_id()
    bufs = (idx0_v, idx1_v)
    edge_base = wid * _E_PER

    def win_src(w):
        return target_hbm.at[pl.ds(edge_base + w * _W, _W)]

    pltpu.async_copy(win_src(0), bufs[0], sems.at[0])
    pltpu.async_copy(win_src(1), bufs[1], sems.at[1])

    @plsc.parallel_loop(0, _NBINS // _LANES, unroll=8)
    def _(i):
        hist_v[pl.ds(i * _LANES, _LANES)] = jnp.zeros((_LANES,), jnp.int32)

    def compute(buf):
        @plsc.parallel_loop(0, _W // _LANES, unroll=8)
        def _(i):
            idx = buf[pl.ds(i * _LANES, _LANES)]
            cnt, last = plsc.scan_count(idx)
            plsc.addupdate_scatter(hist_v, [idx], cnt, mask=last)

    def outer(t, _):
        g = t * 2
        for b in range(2):
            w = g + b
            pltpu.make_async_copy(win_src(w), bufs[b], sems.at[b]).wait()
            compute(bufs[b])

            @pl.when(w + 2 < _NWIN)
            def _():
                pltpu.async_copy(win_src(w + 2), bufs[b], sems.at[b])

        return 0

    lax.fori_loop(0, _NWIN // 2, outer, 0)

    pltpu.sync_copy(hist_v, part_hbm.at[pl.ds(wid * _NBINS, _NBINS)])


_TC_ROWS = _NBINS // 128
_TC_BLK = _TC_ROWS // 4


@functools.partial(
    pl.pallas_call,
    out_shape=jax.ShapeDtypeStruct((_TC_ROWS, 128), jnp.float32),
    grid=(4,),
    in_specs=[pl.BlockSpec((_NW, _TC_BLK, 128), lambda i: (0, i, 0))],
    out_specs=pl.BlockSpec((_TC_BLK, 128), lambda i: (i, 0)),
)
def _reduce_kernel_tc(parts_ref, inv_ref):
    inv_ref[...] = 1.0 / jnp.sum(parts_ref[...], axis=0).astype(jnp.float32)


@functools.partial(
    pl.kernel,
    out_type=jax.ShapeDtypeStruct((_N_EDGES,), jnp.float32),
    mesh=_mesh,
    compiler_params=_params,
    scratch_types=[
        pltpu.VMEM((_NBINS,), jnp.float32),
        pltpu.VMEM((_W,), jnp.int32),
        pltpu.VMEM((_W,), jnp.int32),
        pltpu.VMEM((_W,), jnp.float32),
        pltpu.VMEM((_W,), jnp.float32),
        pltpu.SemaphoreType.DMA((2,)),
        pltpu.SemaphoreType.DMA((2,)),
    ],
)
def _weights_kernel(
    target_hbm, inv_hbm, out_hbm, inv_v, idx0_v, idx1_v, w0_v, w1_v, isems, osems
):
    wid = _worker_id()
    ibufs = (idx0_v, idx1_v)
    obufs = (w0_v, w1_v)

    edge_base = wid * _E_PER

    def win_src(w):
        return target_hbm.at[pl.ds(edge_base + w * _W, _W)]

    def win_dst(w):
        return out_hbm.at[pl.ds(edge_base + w * _W, _W)]

    pltpu.async_copy(win_src(0), ibufs[0], isems.at[0])
    pltpu.async_copy(win_src(1), ibufs[1], isems.at[1])

    pltpu.sync_copy(inv_hbm, inv_v)

    def compute(ibuf, obuf):
        @plsc.parallel_loop(0, _W // _LANES, unroll=4)
        def _(i):
            s = pl.ds(i * _LANES, _LANES)
            obuf[s] = plsc.load_gather(inv_v, [ibuf[s]])

    def outer(t, _):
        g = t * 2
        for b in range(2):
            w = g + b
            pltpu.make_async_copy(win_src(w), ibufs[b], isems.at[b]).wait()

            @pl.when(w >= 2)
            def _():
                pltpu.make_async_copy(obufs[b], win_dst(w - 2), osems.at[b]).wait()

            compute(ibufs[b], obufs[b])
            pltpu.async_copy(obufs[b], win_dst(w), osems.at[b])

            @pl.when(w + 2 < _NWIN)
            def _():
                pltpu.async_copy(win_src(w + 2), ibufs[b], isems.at[b])

        return 0

    lax.fori_loop(0, _NWIN // 2, outer, 0)

    pltpu.make_async_copy(obufs[0], win_dst(_NWIN - 2), osems.at[0]).wait()
    pltpu.make_async_copy(obufs[1], win_dst(_NWIN - 1), osems.at[1]).wait()


def kernel(source, target):
    del source
    target = target.astype(jnp.int32)
    partials = _hist_kernel(target)
    inv_counts = _reduce_kernel_tc(
        partials.reshape(_NW, _TC_ROWS, 128)
    ).reshape(_NBINS)
    weights = _weights_kernel(target, inv_counts)
    return weights


# speedup vs baseline: 1.0802x; 1.0802x over previous
import functools

import jax
import jax.numpy as jnp
from jax import lax
from jax.experimental import pallas as pl
from jax.experimental.pallas import tpu as pltpu
from jax.experimental.pallas import tpu_sc as plsc

_N_NODES = 100000
_N_EDGES = 6400000

_NC = 2
_NS = 16
_NW = _NC * _NS

_NBINS = 102400
_BIN_PER = _NBINS // _NW
_E_PER = _N_EDGES // _NW
_W = 4000
_NWIN = _E_PER // _W
_WH = 8000
_NWINH = _E_PER // _WH
_LANES = 16

_mesh = plsc.VectorSubcoreMesh(
    core_axis_name="c", subcore_axis_name="s", num_cores=_NC, num_subcores=_NS
)

_params = pltpu.CompilerParams(needs_layout_passes=False)


def _worker_id():
    return lax.axis_index("s") * _NC + lax.axis_index("c")


@functools.partial(
    pl.kernel,
    out_type=jax.ShapeDtypeStruct((_NW * _NBINS,), jnp.int32),
    mesh=_mesh,
    compiler_params=_params,
    scratch_types=[
        pltpu.VMEM((_NBINS,), jnp.int32),
        pltpu.VMEM((_WH,), jnp.int32),
        pltpu.VMEM((_WH,), jnp.int32),
        pltpu.SemaphoreType.DMA((2,)),
    ],
)
def _hist_kernel(target_hbm, part_hbm, hist_v, idx0_v, idx1_v, sems):
    wid = _worker_id()
    bufs = (idx0_v, idx1_v)
    edge_base = wid * _E_PER

    def win_src(w):
        return target_hbm.at[pl.ds(edge_base + w * _WH, _WH)]

    pltpu.async_copy(win_src(0), bufs[0], sems.at[0])
    pltpu.async_copy(win_src(1), bufs[1], sems.at[1])

    @plsc.parallel_loop(0, _NBINS // _LANES, unroll=8)
    def _(i):
        hist_v[pl.ds(i * _LANES, _LANES)] = jnp.zeros((_LANES,), jnp.int32)

    def compute(buf):
        @plsc.parallel_loop(0, _WH // _LANES, unroll=8)
        def _(i):
            idx = buf[pl.ds(i * _LANES, _LANES)]
            cnt, last = plsc.scan_count(idx)
            plsc.addupdate_scatter(hist_v, [idx], cnt, mask=last)

    def outer(t, _):
        g = t * 2
        for b in range(2):
            w = g + b
            pltpu.make_async_copy(win_src(w), bufs[b], sems.at[b]).wait()
            compute(bufs[b])

            @pl.when(w + 2 < _NWINH)
            def _():
                pltpu.async_copy(win_src(w + 2), bufs[b], sems.at[b])

        return 0

    lax.fori_loop(0, (_NWINH - 1) // 2, outer, 0)
    pltpu.make_async_copy(win_src(_NWINH - 1), bufs[0], sems.at[0]).wait()
    compute(bufs[0])

    pltpu.sync_copy(hist_v, part_hbm.at[pl.ds(wid * _NBINS, _NBINS)])


_TC_ROWS = _NBINS // 128
_TC_BLK = _TC_ROWS // 4


@functools.partial(
    pl.pallas_call,
    out_shape=jax.ShapeDtypeStruct((_TC_ROWS, 128), jnp.float32),
    grid=(4,),
    in_specs=[pl.BlockSpec((_NW, _TC_BLK, 128), lambda i: (0, i, 0))],
    out_specs=pl.BlockSpec((_TC_BLK, 128), lambda i: (i, 0)),
)
def _reduce_kernel_tc(parts_ref, inv_ref):
    inv_ref[...] = 1.0 / jnp.sum(parts_ref[...], axis=0).astype(jnp.float32)


@functools.partial(
    pl.kernel,
    out_type=jax.ShapeDtypeStruct((_N_EDGES,), jnp.float32),
    mesh=_mesh,
    compiler_params=_params,
    scratch_types=[
        pltpu.VMEM((_NBINS,), jnp.float32),
        pltpu.VMEM((_W,), jnp.int32),
        pltpu.VMEM((_W,), jnp.int32),
        pltpu.VMEM((_W,), jnp.float32),
        pltpu.VMEM((_W,), jnp.float32),
        pltpu.SemaphoreType.DMA((2,)),
        pltpu.SemaphoreType.DMA((2,)),
    ],
)
def _weights_kernel(
    target_hbm, inv_hbm, out_hbm, inv_v, idx0_v, idx1_v, w0_v, w1_v, isems, osems
):
    wid = _worker_id()
    ibufs = (idx0_v, idx1_v)
    obufs = (w0_v, w1_v)

    edge_base = wid * _E_PER

    def win_src(w):
        return target_hbm.at[pl.ds(edge_base + w * _W, _W)]

    def win_dst(w):
        return out_hbm.at[pl.ds(edge_base + w * _W, _W)]

    pltpu.async_copy(win_src(0), ibufs[0], isems.at[0])
    pltpu.async_copy(win_src(1), ibufs[1], isems.at[1])

    pltpu.sync_copy(inv_hbm, inv_v)

    def compute(ibuf, obuf):
        @plsc.parallel_loop(0, _W // _LANES, unroll=8)
        def _(i):
            s = pl.ds(i * _LANES, _LANES)
            obuf[s] = plsc.load_gather(inv_v, [ibuf[s]])

    def outer(t, _):
        g = t * 2
        for b in range(2):
            w = g + b
            pltpu.make_async_copy(win_src(w), ibufs[b], isems.at[b]).wait()

            @pl.when(w >= 2)
            def _():
                pltpu.make_async_copy(obufs[b], win_dst(w - 2), osems.at[b]).wait()

            compute(ibufs[b], obufs[b])
            pltpu.async_copy(obufs[b], win_dst(w), osems.at[b])

            @pl.when(w + 2 < _NWIN)
            def _():
                pltpu.async_copy(win_src(w + 2), ibufs[b], isems.at[b])

        return 0

    lax.fori_loop(0, _NWIN // 2, outer, 0)

    pltpu.make_async_copy(obufs[0], win_dst(_NWIN - 2), osems.at[0]).wait()
    pltpu.make_async_copy(obufs[1], win_dst(_NWIN - 1), osems.at[1]).wait()


def kernel(source, target):
    del source
    target = target.astype(jnp.int32)
    partials = _hist_kernel(target)
    inv_counts = _reduce_kernel_tc(
        partials.reshape(_NW, _TC_ROWS, 128)
    ).reshape(_NBINS)
    weights = _weights_kernel(target, inv_counts)
    return weights


# speedup vs baseline: 1.0958x; 1.0144x over previous
import functools

import jax
import jax.numpy as jnp
from jax import lax
from jax.experimental import pallas as pl
from jax.experimental.pallas import tpu as pltpu
from jax.experimental.pallas import tpu_sc as plsc

_N_NODES = 100000
_N_EDGES = 6400000

_NC = 2
_NS = 16
_NW = _NC * _NS

_NBINS = 102400
_BIN_PER = _NBINS // _NW
_E_PER = _N_EDGES // _NW
_W = 4000
_NWIN = _E_PER // _W
_WH = 8000
_NWINH = _E_PER // _WH
_LANES = 16

_mesh = plsc.VectorSubcoreMesh(
    core_axis_name="c", subcore_axis_name="s", num_cores=_NC, num_subcores=_NS
)

_params = pltpu.CompilerParams(needs_layout_passes=False)


def _worker_id():
    return lax.axis_index("s") * _NC + lax.axis_index("c")


@functools.partial(
    pl.kernel,
    out_type=jax.ShapeDtypeStruct((_NW * _NBINS,), jnp.int32),
    mesh=_mesh,
    compiler_params=_params,
    scratch_types=[
        pltpu.VMEM((_NBINS,), jnp.int32),
        pltpu.VMEM((_WH,), jnp.int32),
        pltpu.VMEM((_WH,), jnp.int32),
        pltpu.SemaphoreType.DMA((2,)),
    ],
)
def _hist_kernel(target_hbm, part_hbm, hist_v, idx0_v, idx1_v, sems):
    wid = _worker_id()
    bufs = (idx0_v, idx1_v)
    edge_base = wid * _E_PER

    def win_src(w):
        return target_hbm.at[pl.ds(edge_base + w * _WH, _WH)]

    pltpu.async_copy(win_src(0), bufs[0], sems.at[0])
    pltpu.async_copy(win_src(1), bufs[1], sems.at[1])

    @plsc.parallel_loop(0, _NBINS // _LANES, unroll=8)
    def _(i):
        hist_v[pl.ds(i * _LANES, _LANES)] = jnp.zeros((_LANES,), jnp.int32)

    def compute(buf):
        @plsc.parallel_loop(0, _WH // _LANES, unroll=8)
        def _(i):
            idx = buf[pl.ds(i * _LANES, _LANES)]
            cnt, last = plsc.scan_count(idx)
            plsc.addupdate_scatter(hist_v, [idx], cnt, mask=last)

    def outer(t, _):
        g = t * 2
        for b in range(2):
            w = g + b
            pltpu.make_async_copy(win_src(w), bufs[b], sems.at[b]).wait()
            compute(bufs[b])

            @pl.when(w + 2 < _NWINH)
            def _():
                pltpu.async_copy(win_src(w + 2), bufs[b], sems.at[b])

        return 0

    lax.fori_loop(0, (_NWINH - 1) // 2, outer, 0)
    pltpu.make_async_copy(win_src(_NWINH - 1), bufs[0], sems.at[0]).wait()
    compute(bufs[0])

    pltpu.sync_copy(hist_v, part_hbm.at[pl.ds(wid * _NBINS, _NBINS)])


_TC_ROWS = _NBINS // 128
_TC_BLK = _TC_ROWS // 4


@functools.partial(
    pl.pallas_call,
    out_shape=jax.ShapeDtypeStruct((_TC_ROWS, 128), jnp.float32),
    grid=(4,),
    in_specs=[pl.BlockSpec((_NW, _TC_BLK, 128), lambda i: (0, i, 0))],
    out_specs=pl.BlockSpec((_TC_BLK, 128), lambda i: (i, 0)),
)
def _reduce_kernel_tc(parts_ref, inv_ref):
    inv_ref[...] = 1.0 / jnp.sum(parts_ref[...], axis=0).astype(jnp.float32)


@functools.partial(
    pl.kernel,
    out_type=jax.ShapeDtypeStruct((_N_EDGES,), jnp.float32),
    mesh=_mesh,
    compiler_params=_params,
    scratch_types=[
        pltpu.VMEM((_NBINS,), jnp.float32),
        pltpu.VMEM((_WH,), jnp.float32),
        pltpu.VMEM((_WH,), jnp.float32),
        pltpu.VMEM((_WH,), jnp.float32),
        pltpu.SemaphoreType.DMA((3,)),
        pltpu.SemaphoreType.DMA((3,)),
    ],
)
def _weights_kernel(targetf_hbm, inv_hbm, out_hbm, inv_v, b0_v, b1_v, b2_v, isems, osems):
    wid = _worker_id()
    bufs = (b0_v, b1_v, b2_v)
    edge_base = wid * _E_PER

    def win_src(w):
        return targetf_hbm.at[pl.ds(edge_base + w * _WH, _WH)]

    def win_dst(w):
        return out_hbm.at[pl.ds(edge_base + w * _WH, _WH)]

    pltpu.async_copy(win_src(0), bufs[0], isems.at[0])
    pltpu.async_copy(win_src(1), bufs[1], isems.at[1])

    pltpu.sync_copy(inv_hbm, inv_v)

    def compute(buf):
        @plsc.parallel_loop(0, _WH // _LANES, unroll=8)
        def _(i):
            s = pl.ds(i * _LANES, _LANES)
            idx = plsc.bitcast(buf[s], jnp.int32)
            buf[s] = plsc.load_gather(inv_v, [idx])

    def step(w, b):
        nb = (b + 2) % 3
        pltpu.make_async_copy(win_src(w), bufs[b], isems.at[b]).wait()
        compute(bufs[b])

        @pl.when(w >= 1)
        def _():
            pltpu.make_async_copy(bufs[nb], win_dst(w - 1), osems.at[nb]).wait()

        @pl.when(w + 2 < _NWINH)
        def _():
            pltpu.async_copy(win_src(w + 2), bufs[nb], isems.at[nb])

        pltpu.async_copy(bufs[b], win_dst(w), osems.at[b])

    def outer(t, _):
        g = t * 3
        for b in range(3):
            step(g + b, b)
        return 0

    lax.fori_loop(0, _NWINH // 3, outer, 0)
    step(_NWINH - 1, (_NWINH - 1) % 3)
    pltpu.make_async_copy(
        bufs[(_NWINH - 1) % 3], win_dst(_NWINH - 1), osems.at[(_NWINH - 1) % 3]
    ).wait()


def kernel(source, target):
    del source
    target = target.astype(jnp.int32)
    partials = _hist_kernel(target)
    inv_counts = _reduce_kernel_tc(
        partials.reshape(_NW, _TC_ROWS, 128)
    ).reshape(_NBINS)
    targetf = lax.bitcast_convert_type(target, jnp.float32)
    weights = _weights_kernel(targetf, inv_counts)
    return weights


# speedup vs baseline: 1.1160x; 1.0185x over previous
import functools

import jax
import jax.numpy as jnp
from jax import lax
from jax.experimental import pallas as pl
from jax.experimental.pallas import tpu as pltpu
from jax.experimental.pallas import tpu_sc as plsc

_N_NODES = 100000
_N_EDGES = 6400000

_NC = 2
_NS = 16
_NW = _NC * _NS

_NBINS = 102400
_BIN_PER = _NBINS // _NW
_E_PER = _N_EDGES // _NW
_W = 4000
_NWIN = _E_PER // _W
_WH = 8000
_NWINH = _E_PER // _WH
_WB = 10000
_NWINB = _E_PER // _WB
_LANES = 16

_mesh = plsc.VectorSubcoreMesh(
    core_axis_name="c", subcore_axis_name="s", num_cores=_NC, num_subcores=_NS
)

_params = pltpu.CompilerParams(needs_layout_passes=False)


def _worker_id():
    return lax.axis_index("s") * _NC + lax.axis_index("c")


@functools.partial(
    pl.kernel,
    out_type=jax.ShapeDtypeStruct((_NW * _NBINS,), jnp.int32),
    mesh=_mesh,
    compiler_params=_params,
    scratch_types=[
        pltpu.VMEM((_NBINS,), jnp.int32),
        pltpu.VMEM((_WB,), jnp.int32),
        pltpu.VMEM((_WB,), jnp.int32),
        pltpu.SemaphoreType.DMA((2,)),
    ],
)
def _hist_kernel(target_hbm, part_hbm, hist_v, idx0_v, idx1_v, sems):
    wid = _worker_id()
    bufs = (idx0_v, idx1_v)
    edge_base = wid * _E_PER

    def win_src(w):
        return target_hbm.at[pl.ds(edge_base + w * _WB, _WB)]

    pltpu.async_copy(win_src(0), bufs[0], sems.at[0])
    pltpu.async_copy(win_src(1), bufs[1], sems.at[1])

    @plsc.parallel_loop(0, _NBINS // _LANES, unroll=8)
    def _(i):
        hist_v[pl.ds(i * _LANES, _LANES)] = jnp.zeros((_LANES,), jnp.int32)

    def compute(buf):
        @plsc.parallel_loop(0, _WB // _LANES, unroll=8)
        def _(i):
            idx = buf[pl.ds(i * _LANES, _LANES)]
            cnt, last = plsc.scan_count(idx)
            plsc.addupdate_scatter(hist_v, [idx], cnt, mask=last)

    def outer(t, _):
        g = t * 2
        for b in range(2):
            w = g + b
            pltpu.make_async_copy(win_src(w), bufs[b], sems.at[b]).wait()
            compute(bufs[b])

            @pl.when(w + 2 < _NWINB)
            def _():
                pltpu.async_copy(win_src(w + 2), bufs[b], sems.at[b])

        return 0

    lax.fori_loop(0, _NWINB // 2, outer, 0)

    pltpu.sync_copy(hist_v, part_hbm.at[pl.ds(wid * _NBINS, _NBINS)])


_TC_ROWS = _NBINS // 128
_TC_BLK = _TC_ROWS // 4


@functools.partial(
    pl.pallas_call,
    out_shape=jax.ShapeDtypeStruct((_TC_ROWS, 128), jnp.float32),
    grid=(4,),
    in_specs=[pl.BlockSpec((_NW, _TC_BLK, 128), lambda i: (0, i, 0))],
    out_specs=pl.BlockSpec((_TC_BLK, 128), lambda i: (i, 0)),
)
def _reduce_kernel_tc(parts_ref, inv_ref):
    inv_ref[...] = 1.0 / jnp.sum(parts_ref[...], axis=0).astype(jnp.float32)


@functools.partial(
    pl.kernel,
    out_type=jax.ShapeDtypeStruct((_N_EDGES,), jnp.float32),
    mesh=_mesh,
    compiler_params=_params,
    scratch_types=[
        pltpu.VMEM((_NBINS,), jnp.float32),
        pltpu.VMEM((_WH,), jnp.float32),
        pltpu.VMEM((_WH,), jnp.float32),
        pltpu.VMEM((_WH,), jnp.float32),
        pltpu.SemaphoreType.DMA((3,)),
        pltpu.SemaphoreType.DMA((3,)),
    ],
)
def _weights_kernel(targetf_hbm, inv_hbm, out_hbm, inv_v, b0_v, b1_v, b2_v, isems, osems):
    wid = _worker_id()
    bufs = (b0_v, b1_v, b2_v)
    edge_base = wid * _E_PER

    def win_src(w):
        return targetf_hbm.at[pl.ds(edge_base + w * _WH, _WH)]

    def win_dst(w):
        return out_hbm.at[pl.ds(edge_base + w * _WH, _WH)]

    pltpu.async_copy(win_src(0), bufs[0], isems.at[0])
    pltpu.async_copy(win_src(1), bufs[1], isems.at[1])

    pltpu.sync_copy(inv_hbm, inv_v)

    def compute(buf):
        @plsc.parallel_loop(0, _WH // _LANES, unroll=8)
        def _(i):
            s = pl.ds(i * _LANES, _LANES)
            idx = plsc.bitcast(buf[s], jnp.int32)
            buf[s] = plsc.load_gather(inv_v, [idx])

    def step(w, b):
        nb = (b + 2) % 3
        pltpu.make_async_copy(win_src(w), bufs[b], isems.at[b]).wait()
        compute(bufs[b])

        @pl.when(w >= 1)
        def _():
            pltpu.make_async_copy(bufs[nb], win_dst(w - 1), osems.at[nb]).wait()

        @pl.when(w + 2 < _NWINH)
        def _():
            pltpu.async_copy(win_src(w + 2), bufs[nb], isems.at[nb])

        pltpu.async_copy(bufs[b], win_dst(w), osems.at[b])

    def outer(t, _):
        g = t * 3
        for b in range(3):
            step(g + b, b)
        return 0

    lax.fori_loop(0, _NWINH // 3, outer, 0)
    step(_NWINH - 1, (_NWINH - 1) % 3)
    pltpu.make_async_copy(
        bufs[(_NWINH - 1) % 3], win_dst(_NWINH - 1), osems.at[(_NWINH - 1) % 3]
    ).wait()


def kernel(source, target):
    del source
    target = target.astype(jnp.int32)
    partials = _hist_kernel(target)
    inv_counts = _reduce_kernel_tc(
        partials.reshape(_NW, _TC_ROWS, 128)
    ).reshape(_NBINS)
    targetf = lax.bitcast_convert_type(target, jnp.float32)
    weights = _weights_kernel(targetf, inv_counts)
    return weights


# speedup vs baseline: 1.1229x; 1.0061x over previous
import functools

import jax
import jax.numpy as jnp
from jax import lax
from jax.experimental import pallas as pl
from jax.experimental.pallas import tpu as pltpu
from jax.experimental.pallas import tpu_sc as plsc

_N_NODES = 100000
_N_EDGES = 6400000

_NC = 2
_NS = 16
_NW = _NC * _NS

_NBINS = 102400
_BIN_PER = _NBINS // _NW
_E_PER = _N_EDGES // _NW
_W = 4000
_NWIN = _E_PER // _W
_WH = 8000
_NWINH = _E_PER // _WH
_WB = 10000
_NWINB = _E_PER // _WB
_LANES = 16

_mesh = plsc.VectorSubcoreMesh(
    core_axis_name="c", subcore_axis_name="s", num_cores=_NC, num_subcores=_NS
)

_params = pltpu.CompilerParams(needs_layout_passes=False)


def _worker_id():
    return lax.axis_index("s") * _NC + lax.axis_index("c")


@functools.partial(
    pl.kernel,
    out_type=jax.ShapeDtypeStruct((_NW * _NBINS,), jnp.int32),
    mesh=_mesh,
    compiler_params=_params,
    scratch_types=[
        pltpu.VMEM((_NBINS,), jnp.int32),
        pltpu.VMEM((_WB,), jnp.int32),
        pltpu.VMEM((_WB,), jnp.int32),
        pltpu.SemaphoreType.DMA((2,)),
    ],
)
def _hist_kernel(target_hbm, part_hbm, hist_v, idx0_v, idx1_v, sems):
    wid = _worker_id()
    bufs = (idx0_v, idx1_v)
    edge_base = wid * _E_PER

    def win_src(w):
        return target_hbm.at[pl.ds(edge_base + w * _WB, _WB)]

    pltpu.async_copy(win_src(0), bufs[0], sems.at[0])
    pltpu.async_copy(win_src(1), bufs[1], sems.at[1])

    @plsc.parallel_loop(0, _NBINS // _LANES, unroll=8)
    def _(i):
        hist_v[pl.ds(i * _LANES, _LANES)] = jnp.zeros((_LANES,), jnp.int32)

    def compute(buf):
        @plsc.parallel_loop(0, _WB // _LANES, unroll=8)
        def _(i):
            idx = buf[pl.ds(i * _LANES, _LANES)]
            cnt, last = plsc.scan_count(idx)
            plsc.addupdate_scatter(hist_v, [idx], cnt, mask=last)

    def outer(t, _):
        g = t * 2
        for b in range(2):
            w = g + b
            pltpu.make_async_copy(win_src(w), bufs[b], sems.at[b]).wait()
            compute(bufs[b])

            @pl.when(w + 2 < _NWINB)
            def _():
                pltpu.async_copy(win_src(w + 2), bufs[b], sems.at[b])

        return 0

    lax.fori_loop(0, _NWINB // 2, outer, 0)

    pltpu.sync_copy(hist_v, part_hbm.at[pl.ds(wid * _NBINS, _NBINS)])


_TC_ROWS = _NBINS // 128
_TC_BLK = _TC_ROWS // 2


@functools.partial(
    pl.pallas_call,
    out_shape=jax.ShapeDtypeStruct((_TC_ROWS, 128), jnp.float32),
    grid=(2,),
    in_specs=[pl.BlockSpec((_NW, _TC_BLK, 128), lambda i: (0, i, 0))],
    out_specs=pl.BlockSpec((_TC_BLK, 128), lambda i: (i, 0)),
)
def _reduce_kernel_tc(parts_ref, inv_ref):
    inv_ref[...] = 1.0 / jnp.sum(parts_ref[...], axis=0).astype(jnp.float32)


@functools.partial(
    pl.kernel,
    out_type=jax.ShapeDtypeStruct((_N_EDGES,), jnp.float32),
    mesh=_mesh,
    compiler_params=_params,
    scratch_types=[
        pltpu.VMEM((_NBINS,), jnp.float32),
        pltpu.VMEM((_WH,), jnp.float32),
        pltpu.VMEM((_WH,), jnp.float32),
        pltpu.VMEM((_WH,), jnp.float32),
        pltpu.SemaphoreType.DMA((3,)),
        pltpu.SemaphoreType.DMA((3,)),
    ],
)
def _weights_kernel(targetf_hbm, inv_hbm, out_hbm, inv_v, b0_v, b1_v, b2_v, isems, osems):
    wid = _worker_id()
    bufs = (b0_v, b1_v, b2_v)
    edge_base = wid * _E_PER

    def win_src(w):
        return targetf_hbm.at[pl.ds(edge_base + w * _WH, _WH)]

    def win_dst(w):
        return out_hbm.at[pl.ds(edge_base + w * _WH, _WH)]

    pltpu.async_copy(win_src(0), bufs[0], isems.at[0])
    pltpu.async_copy(win_src(1), bufs[1], isems.at[1])

    pltpu.sync_copy(inv_hbm, inv_v)

    def compute(buf):
        @plsc.parallel_loop(0, _WH // _LANES, unroll=8)
        def _(i):
            s = pl.ds(i * _LANES, _LANES)
            idx = plsc.bitcast(buf[s], jnp.int32)
            buf[s] = plsc.load_gather(inv_v, [idx])

    def step(w, b):
        nb = (b + 2) % 3
        pltpu.make_async_copy(win_src(w), bufs[b], isems.at[b]).wait()
        compute(bufs[b])

        @pl.when(w >= 1)
        def _():
            pltpu.make_async_copy(bufs[nb], win_dst(w - 1), osems.at[nb]).wait()

        @pl.when(w + 2 < _NWINH)
        def _():
            pltpu.async_copy(win_src(w + 2), bufs[nb], isems.at[nb])

        pltpu.async_copy(bufs[b], win_dst(w), osems.at[b])

    def outer(t, _):
        g = t * 3
        for b in range(3):
            step(g + b, b)
        return 0

    lax.fori_loop(0, _NWINH // 3, outer, 0)
    step(_NWINH - 1, (_NWINH - 1) % 3)
    pltpu.make_async_copy(
        bufs[(_NWINH - 1) % 3], win_dst(_NWINH - 1), osems.at[(_NWINH - 1) % 3]
    ).wait()


def kernel(source, target):
    del source
    target = target.astype(jnp.int32)
    partials = _hist_kernel(target)
    inv_counts = _reduce_kernel_tc(
        partials.reshape(_NW, _TC_ROWS, 128)
    ).reshape(_NBINS)
    targetf = lax.bitcast_convert_type(target, jnp.float32)
    weights = _weights_kernel(targetf, inv_counts)
    return weights
